# baseline (speedup 1.0000x reference)
import functools

import jax
import jax.numpy as jnp
from jax import lax
from jax.experimental import pallas as pl
from jax.experimental.pallas import tpu as pltpu
from jax.experimental.pallas import tpu_sc as plsc

B, T, D = 4096, 50, 128
NC, NS = 2, 16
NW = NC * NS
BPW = B // NW
CB = 8
NSUB = BPW // CB


def _sc_body(feat_hbm, idx_hbm, table_hbm, out_hbm,
             idx_v, buf0, buf1,
             gsem, fsem0, fsem1, osem0, osem1):
    wid = lax.axis_index("s") * NC + lax.axis_index("c")
    base = wid * BPW

    pltpu.sync_copy(idx_hbm.at[pl.ds(base, BPW)], idx_v)

    bufs = (buf0, buf1)
    fsems = (fsem0, fsem1)
    osems = (osem0, osem1)

    def fcopy(i, p):
        return pltpu.make_async_copy(
            feat_hbm.at[pl.ds(base + i * CB, CB)],
            bufs[p].at[:, pl.ds(1, T)],
            fsems[p])

    def ocopy(i, p):
        return pltpu.make_async_copy(
            bufs[p],
            out_hbm.at[pl.ds(base + i * CB, CB)],
            osems[p])

    fcopy(0, 0).start()
    for i in range(NSUB):
        p = i & 1
        if i + 1 < NSUB:
            if i >= 1:
                ocopy(i - 1, 1 - p).wait()
            fcopy(i + 1, 1 - p).start()
        fcopy(i, p).wait()
        pltpu.async_copy(
            table_hbm.at[idx_v.at[pl.ds(i * CB, CB)]],
            bufs[p].at[:, 0], gsem).wait()
        ocopy(i, p).start()
    ocopy(NSUB - 2, (NSUB - 2) & 1).wait()
    ocopy(NSUB - 1, (NSUB - 1) & 1).wait()


@jax.jit
def _run(feature, idx, table):
    mesh = plsc.VectorSubcoreMesh(core_axis_name="c", subcore_axis_name="s")
    fn = functools.partial(
        pl.kernel,
        out_type=jax.ShapeDtypeStruct((B, T + 1, D), jnp.float32),
        mesh=mesh,
        scratch_types=[
            pltpu.VMEM((BPW,), jnp.int32),
            pltpu.VMEM((CB, T + 1, D), jnp.float32),
            pltpu.VMEM((CB, T + 1, D), jnp.float32),
            pltpu.SemaphoreType.DMA,
            pltpu.SemaphoreType.DMA,
            pltpu.SemaphoreType.DMA,
            pltpu.SemaphoreType.DMA,
            pltpu.SemaphoreType.DMA,
        ],
    )(_sc_body)
    return fn(feature, idx, table)


def kernel(feature, index_value, embedding_table):
    idx = index_value.astype(jnp.int32)
    return _run(feature, idx, embedding_table)


# speedup vs baseline: 1.0440x; 1.0440x over previous
import functools

import jax
import jax.numpy as jnp
from jax import lax
from jax.experimental import pallas as pl
from jax.experimental.pallas import tpu as pltpu
from jax.experimental.pallas import tpu_sc as plsc

B, T, D = 4096, 50, 128
NC, NS = 2, 16
NW = NC * NS
BPW = B // NW

C = 64
NCH = B // C
K = 8


def _sc_gather_body(idx_hbm, table_hbm, emb_hbm, idx_v, rows_v, sem):
    wid = lax.axis_index("s") * NC + lax.axis_index("c")
    base = wid * BPW
    pltpu.sync_copy(idx_hbm.at[pl.ds(base, BPW)], idx_v)
    pltpu.async_copy(table_hbm.at[idx_v], rows_v, sem).wait()
    pltpu.sync_copy(rows_v, emb_hbm.at[pl.ds(base, BPW)])


def _tc_concat_body(emb_hbm, feat_hbm, out_hbm,
                    feat_buf, emb_buf, out_buf,
                    in_sems, emb_sems, out_sems):
    def in_copies(g, slot):
        return (
            pltpu.make_async_copy(
                feat_hbm.at[pl.ds(g * C, C)], feat_buf.at[slot],
                in_sems.at[slot]),
            pltpu.make_async_copy(
                emb_hbm.at[pl.ds(g * C, C)], emb_buf.at[slot],
                emb_sems.at[slot]),
        )

    def out_copy(g, slot):
        return pltpu.make_async_copy(
            out_buf.at[slot], out_hbm.at[pl.ds(g * C, C)],
            out_sems.at[slot])

    for g in range(K):
        for c in in_copies(g, g):
            c.start()

    for g in range(NCH):
        slot = g % K
        for c in in_copies(g, slot):
            c.wait()
        if g >= K:
            out_copy(g - K, slot).wait()
        out_buf[slot, :, 0, :] = emb_buf[slot]
        out_buf[slot, :, 1:, :] = feat_buf[slot]
        out_copy(g, slot).start()
        if g + K < NCH:
            for c in in_copies(g + K, slot):
                c.start()

    for t in range(NCH - K, NCH):
        out_copy(t, t % K).wait()


@jax.jit
def _run(feature, idx, table):
    mesh = plsc.VectorSubcoreMesh(core_axis_name="c", subcore_axis_name="s")
    emb = functools.partial(
        pl.kernel,
        out_type=jax.ShapeDtypeStruct((B, D), jnp.float32),
        mesh=mesh,
        scratch_types=[
            pltpu.VMEM((BPW,), jnp.int32),
            pltpu.VMEM((BPW, D), jnp.float32),
            pltpu.SemaphoreType.DMA,
        ],
    )(_sc_gather_body)(idx, table)

    return pl.pallas_call(
        _tc_concat_body,
        in_specs=[
            pl.BlockSpec(memory_space=pl.ANY),
            pl.BlockSpec(memory_space=pl.ANY),
        ],
        out_specs=pl.BlockSpec(memory_space=pl.ANY),
        out_shape=jax.ShapeDtypeStruct((B, T + 1, D), jnp.float32),
        scratch_shapes=[
            pltpu.VMEM((K, C, T, D), jnp.float32),
            pltpu.VMEM((K, C, D), jnp.float32),
            pltpu.VMEM((K, C, T + 1, D), jnp.float32),
            pltpu.SemaphoreType.DMA((K,)),
            pltpu.SemaphoreType.DMA((K,)),
            pltpu.SemaphoreType.DMA((K,)),
        ],
    )(emb, feature)


def kernel(feature, index_value, embedding_table):
    idx = index_value.astype(jnp.int32)
    return _run(feature, idx, embedding_table)


# speedup vs baseline: 1.0475x; 1.0034x over previous
import functools

import jax
import jax.numpy as jnp
from jax import lax
from jax.experimental import pallas as pl
from jax.experimental.pallas import tpu as pltpu
from jax.experimental.pallas import tpu_sc as plsc

B, T, D = 4096, 50, 128
NC, NS = 2, 16
NW = NC * NS
BPW = B // NW

C = 128
NCH = B // C
K = 4


def _sc_gather_body(idx_hbm, table_hbm, emb_hbm, idx_v, rows_v, sem):
    wid = lax.axis_index("s") * NC + lax.axis_index("c")
    base = wid * BPW
    pltpu.sync_copy(idx_hbm.at[pl.ds(base, BPW)], idx_v)
    pltpu.async_copy(table_hbm.at[idx_v], rows_v, sem).wait()
    pltpu.sync_copy(rows_v, emb_hbm.at[pl.ds(base, BPW)])


def _tc_concat_body(emb_hbm, feat_hbm, out_hbm,
                    feat_buf, emb_buf, out_buf,
                    in_sems, emb_sems, out_sems):
    def in_copies(g, slot):
        return (
            pltpu.make_async_copy(
                feat_hbm.at[pl.ds(g * C, C)], feat_buf.at[slot],
                in_sems.at[slot]),
            pltpu.make_async_copy(
                emb_hbm.at[pl.ds(g * C, C)], emb_buf.at[slot],
                emb_sems.at[slot]),
        )

    def out_copy(g, slot):
        return pltpu.make_async_copy(
            out_buf.at[slot], out_hbm.at[pl.ds(g * C, C)],
            out_sems.at[slot])

    for g in range(K):
        for c in in_copies(g, g):
            c.start()

    for g in range(NCH):
        slot = g % K
        for c in in_copies(g, slot):
            c.wait()
        if g >= K:
            out_copy(g - K, slot).wait()
        out_buf[slot, :, 0, :] = emb_buf[slot]
        out_buf[slot, :, 1:, :] = feat_buf[slot]
        out_copy(g, slot).start()
        if g + K < NCH:
            for c in in_copies(g + K, slot):
                c.start()

    for t in range(NCH - K, NCH):
        out_copy(t, t % K).wait()


@jax.jit
def _run(feature, idx, table):
    mesh = plsc.VectorSubcoreMesh(core_axis_name="c", subcore_axis_name="s")
    emb = functools.partial(
        pl.kernel,
        out_type=jax.ShapeDtypeStruct((B, D), jnp.float32),
        mesh=mesh,
        scratch_types=[
            pltpu.VMEM((BPW,), jnp.int32),
            pltpu.VMEM((BPW, D), jnp.float32),
            pltpu.SemaphoreType.DMA,
        ],
    )(_sc_gather_body)(idx, table)

    return pl.pallas_call(
        _tc_concat_body,
        in_specs=[
            pl.BlockSpec(memory_space=pl.ANY),
            pl.BlockSpec(memory_space=pl.ANY),
        ],
        out_specs=pl.BlockSpec(memory_space=pl.ANY),
        out_shape=jax.ShapeDtypeStruct((B, T + 1, D), jnp.float32),
        scratch_shapes=[
            pltpu.VMEM((K, C, T, D), jnp.float32),
            pltpu.VMEM((K, C, D), jnp.float32),
            pltpu.VMEM((K, C, T + 1, D), jnp.float32),
            pltpu.SemaphoreType.DMA((K,)),
            pltpu.SemaphoreType.DMA((K,)),
            pltpu.SemaphoreType.DMA((K,)),
        ],
    )(emb, feature)


def kernel(feature, index_value, embedding_table):
    idx = index_value.astype(jnp.int32)
    return _run(feature, idx, embedding_table)
